# baseline (speedup 1.0000x reference)
import jax
import jax.numpy as jnp
from jax.experimental import pallas as pl
from jax.experimental.pallas import tpu as pltpu

EPS = 1e-10
B = 128
V = 100000
BV = 5000
NV = (V + BV - 1) // BV


def _body(lt_ref, ut_ref, out_ref, m_ref, idx_ref):
    i = pl.program_id(0)

    @pl.when(i < NV)
    def _reduce():
        z = lt_ref[...] - jnp.log(-jnp.log(ut_ref[...] + EPS) + EPS)
        row = jax.lax.broadcasted_iota(jnp.int32, z.shape, 0) + i * BV
        z = jnp.where(row < V, z, -jnp.inf)
        bmax = jnp.max(z, axis=0, keepdims=True)
        bidx = jnp.min(jnp.where(z == bmax, row, V), axis=0, keepdims=True)

        @pl.when(i == 0)
        def _():
            m_ref[...] = bmax
            idx_ref[...] = bidx

        @pl.when(i != 0)
        def _():
            better = bmax > m_ref[...]
            m_ref[...] = jnp.where(better, bmax, m_ref[...])
            idx_ref[...] = jnp.where(better, bidx, idx_ref[...])

    @pl.when(i >= NV)
    def _write():
        row = jax.lax.broadcasted_iota(jnp.int32, (BV, B), 0) + (i - NV) * BV
        out_ref[...] = (row == idx_ref[...]).astype(jnp.float32)


def kernel(logits, u):
    out_t = pl.pallas_call(
        _body,
        grid=(2 * NV,),
        in_specs=[
            pl.BlockSpec((BV, B), lambda i: (jnp.minimum(i, NV - 1), 0)),
            pl.BlockSpec((BV, B), lambda i: (jnp.minimum(i, NV - 1), 0)),
        ],
        out_specs=pl.BlockSpec((BV, B), lambda i: (jnp.maximum(i - NV, 0), 0)),
        out_shape=jax.ShapeDtypeStruct((V, B), jnp.float32),
        scratch_shapes=[
            pltpu.VMEM((1, B), jnp.float32),
            pltpu.VMEM((1, B), jnp.int32),
        ],
    )(logits.T, u.T)
    return out_t.T


# speedup vs baseline: 1.0899x; 1.0899x over previous
import jax
import jax.numpy as jnp
from jax.experimental import pallas as pl
from jax.experimental.pallas import tpu as pltpu

EPS = 1e-10
B = 128
V = 100000
BV = 10000
NV = (V + BV - 1) // BV


def _body(lt_ref, ut_ref, out_ref, m_ref, idx_ref):
    i = pl.program_id(0)

    @pl.when(i < NV)
    def _reduce():
        z = lt_ref[...] - jnp.log(-jnp.log(ut_ref[...] + EPS) + EPS)
        row = jax.lax.broadcasted_iota(jnp.int32, z.shape, 0) + i * BV
        z = jnp.where(row < V, z, -jnp.inf)
        bmax = jnp.max(z, axis=0, keepdims=True)
        bidx = jnp.min(jnp.where(z == bmax, row, V), axis=0, keepdims=True)

        @pl.when(i == 0)
        def _():
            m_ref[...] = bmax
            idx_ref[...] = bidx

        @pl.when(i != 0)
        def _():
            better = bmax > m_ref[...]
            m_ref[...] = jnp.where(better, bmax, m_ref[...])
            idx_ref[...] = jnp.where(better, bidx, idx_ref[...])

    @pl.when(i >= NV)
    def _write():
        row = jax.lax.broadcasted_iota(jnp.int32, (BV, B), 0) + (i - NV) * BV
        out_ref[...] = (row == idx_ref[...]).astype(jnp.float32)


def kernel(logits, u):
    out_t = pl.pallas_call(
        _body,
        grid=(2 * NV,),
        in_specs=[
            pl.BlockSpec((BV, B), lambda i: (jnp.minimum(i, NV - 1), 0)),
            pl.BlockSpec((BV, B), lambda i: (jnp.minimum(i, NV - 1), 0)),
        ],
        out_specs=pl.BlockSpec((BV, B), lambda i: (jnp.maximum(i - NV, 0), 0)),
        out_shape=jax.ShapeDtypeStruct((V, B), jnp.float32),
        scratch_shapes=[
            pltpu.VMEM((1, B), jnp.float32),
            pltpu.VMEM((1, B), jnp.int32),
        ],
    )(logits.T, u.T)
    return out_t.T


# speedup vs baseline: 1.1459x; 1.0514x over previous
import jax
import jax.numpy as jnp
from jax.experimental import pallas as pl
from jax.experimental.pallas import tpu as pltpu

EPS = 1e-10
B = 128
V = 100000
BV = 10000
NV = (V + BV - 1) // BV


def _body(lt_ref, ut_ref, out_ref, m_ref, idx_ref):
    i = pl.program_id(0)

    @pl.when(i < NV)
    def _reduce():
        z = lt_ref[...] - jnp.log(-jnp.log(ut_ref[...] + EPS) + EPS)
        row = jax.lax.broadcasted_iota(jnp.int32, z.shape, 0) + i * BV
        bmax = jnp.max(z, axis=0, keepdims=True)
        bidx = jnp.min(jnp.where(z == bmax, row, V), axis=0, keepdims=True)

        @pl.when(i == 0)
        def _():
            m_ref[...] = bmax
            idx_ref[...] = bidx

        @pl.when(i != 0)
        def _():
            better = bmax > m_ref[...]
            m_ref[...] = jnp.where(better, bmax, m_ref[...])
            idx_ref[...] = jnp.where(better, bidx, idx_ref[...])

    @pl.when(i >= NV)
    def _write():
        row = jax.lax.broadcasted_iota(jnp.int32, (BV, B), 0) + (i - NV) * BV
        out_ref[...] = (row == idx_ref[...]).astype(jnp.float32)


def kernel(logits, u):
    out_t = pl.pallas_call(
        _body,
        grid=(2 * NV,),
        in_specs=[
            pl.BlockSpec((BV, B), lambda i: (jnp.minimum(i, NV - 1), 0)),
            pl.BlockSpec((BV, B), lambda i: (jnp.minimum(i, NV - 1), 0)),
        ],
        out_specs=pl.BlockSpec((BV, B), lambda i: (jnp.maximum(i - NV, 0), 0)),
        out_shape=jax.ShapeDtypeStruct((V, B), jnp.float32),
        scratch_shapes=[
            pltpu.VMEM((1, B), jnp.float32),
            pltpu.VMEM((1, B), jnp.int32),
        ],
    )(logits.T, u.T)
    return out_t.T
